# Initial kernel scaffold; baseline (speedup 1.0000x reference)
import functools

import jax
import jax.numpy as jnp
from jax import lax
from jax.experimental import pallas as pl
from jax.experimental.pallas import tpu as pltpu
from jax.experimental.pallas import tpu_sc as plsc

_NC = 2
_NS = 16
_K = 100


@functools.partial(jax.jit, static_argnames=("n_nodes", "channels"))
def _spmm_partials(h, src2d, dst2d, zeros, *, n_nodes, channels):
    nrows_total = src2d.shape[0]
    ntiles = _NC * _NS
    nct = nrows_total // ntiles
    rpt = n_nodes // _NS
    mesh = plsc.VectorSubcoreMesh(core_axis_name="c", subcore_axis_name="s")

    @functools.partial(
        pl.kernel,
        out_type=jax.ShapeDtypeStruct((2 * n_nodes, channels), jnp.float32),
        mesh=mesh,
        scratch_types=[
            pltpu.VMEM((nct, _K), jnp.int32),
            pltpu.VMEM((nct, _K), jnp.int32),
            pltpu.VMEM((_K, channels), jnp.float32),
            pltpu.VMEM_SHARED((n_nodes, channels), jnp.float32),
            pltpu.SemaphoreType.DMA,
        ],
    )
    def spmm(h_hbm, src_hbm, dst_hbm, zeros_hbm, out_hbm,
             src_v, dst_v, rows_v, acc, sem):
        cid = lax.axis_index("c")
        sid = lax.axis_index("s")
        tid = cid * _NS + sid
        pltpu.sync_copy(src_hbm.at[pl.ds(tid * nct, nct)], src_v)
        pltpu.sync_copy(dst_hbm.at[pl.ds(tid * nct, nct)], dst_v)
        pltpu.sync_copy(zeros_hbm.at[pl.ds(sid * rpt, rpt)],
                        acc.at[pl.ds(sid * rpt, rpt)])
        plsc.subcore_barrier()

        def body(j, carry):
            pltpu.async_copy(h_hbm.at[src_v.at[j]], rows_v, sem).wait()
            pltpu.sync_copy(rows_v, acc.at[dst_v.at[j]], add=True)
            return carry

        lax.fori_loop(0, nct, body, 0)
        plsc.subcore_barrier()
        pltpu.sync_copy(acc.at[pl.ds(sid * rpt, rpt)],
                        out_hbm.at[pl.ds(cid * n_nodes + sid * rpt, rpt)])

    return spmm(h, src2d, dst2d, zeros)


def _mm2_body(p0, p1, w0, w1, o):
    s = p0[...] + p1[...]
    hid = jnp.maximum(jnp.dot(s, w0[...], preferred_element_type=jnp.float32), 0.0)
    o[...] = jnp.dot(hid, w1[...], preferred_element_type=jnp.float32)


def _mm1_body(q0, q1, w2, o):
    s = jnp.maximum(q0[...] + q1[...], 0.0)
    o[...] = jnp.dot(s, w2[...], preferred_element_type=jnp.float32)


def _softmax_body(r0, r1, o):
    s = (r0[...] + r1[...])[:, :40]
    m = jnp.max(s, axis=-1, keepdims=True)
    e = jnp.exp(s - m)
    o[...] = e / jnp.sum(e, axis=-1, keepdims=True)


def kernel(x, edge_index, W0, W1, W2):
    n, d_feat = x.shape
    e = edge_index.shape[1]
    c0 = W0.shape[1]
    c1 = W1.shape[1]
    ncls = W2.shape[1]
    ncls_pad = 48

    src2d = edge_index[0].reshape(e // _K, _K)
    dst2d = edge_index[1].reshape(e // _K, _K)
    zeros_f = jnp.zeros((n, d_feat), jnp.float32)
    zeros_c = jnp.zeros((n, ncls_pad), jnp.float32)
    W2p = jnp.pad(W2, ((0, 0), (0, ncls_pad - ncls)))

    blk = 1000
    grid = (n // blk,)

    agg0 = _spmm_partials(x, src2d, dst2d, zeros_f,
                          n_nodes=n, channels=d_feat)

    t1 = pl.pallas_call(
        _mm2_body,
        grid=grid,
        in_specs=[
            pl.BlockSpec((blk, d_feat), lambda i: (i, 0)),
            pl.BlockSpec((blk, d_feat), lambda i: (i, 0)),
            pl.BlockSpec((d_feat, c0), lambda i: (0, 0)),
            pl.BlockSpec((c0, c1), lambda i: (0, 0)),
        ],
        out_specs=pl.BlockSpec((blk, c1), lambda i: (i, 0)),
        out_shape=jax.ShapeDtypeStruct((n, c1), jnp.float32),
    )(agg0[:n], agg0[n:], W0, W1)

    agg1 = _spmm_partials(t1, src2d, dst2d, zeros_f,
                          n_nodes=n, channels=c1)

    t2 = pl.pallas_call(
        _mm1_body,
        grid=grid,
        in_specs=[
            pl.BlockSpec((blk, c1), lambda i: (i, 0)),
            pl.BlockSpec((blk, c1), lambda i: (i, 0)),
            pl.BlockSpec((c1, ncls_pad), lambda i: (0, 0)),
        ],
        out_specs=pl.BlockSpec((blk, ncls_pad), lambda i: (i, 0)),
        out_shape=jax.ShapeDtypeStruct((n, ncls_pad), jnp.float32),
    )(agg1[:n], agg1[n:], W2p)

    agg2 = _spmm_partials(t2, src2d, dst2d, zeros_c,
                          n_nodes=n, channels=ncls_pad)

    out = pl.pallas_call(
        _softmax_body,
        grid=grid,
        in_specs=[
            pl.BlockSpec((blk, ncls_pad), lambda i: (i, 0)),
            pl.BlockSpec((blk, ncls_pad), lambda i: (i, 0)),
        ],
        out_specs=pl.BlockSpec((blk, ncls), lambda i: (i, 0)),
        out_shape=jax.ShapeDtypeStruct((n, ncls), jnp.float32),
    )(agg2[:n], agg2[n:])

    return out


# speedup vs baseline: 9.0357x; 9.0357x over previous
import functools

import jax
import jax.numpy as jnp
from jax import lax
from jax.experimental import pallas as pl
from jax.experimental.pallas import tpu as pltpu
from jax.experimental.pallas import tpu_sc as plsc

_NC = 2
_NS = 16
_K = 125


@functools.partial(jax.jit, static_argnames=("n_nodes", "channels"))
def _spmm_partials(h, src2d, dst2d, zeros, *, n_nodes, channels):
    nrows_total = src2d.shape[0]
    np_nodes = zeros.shape[0]
    ntiles = _NC * _NS
    nct = nrows_total // ntiles
    rpt = np_nodes // _NS
    mesh = plsc.VectorSubcoreMesh(core_axis_name="c", subcore_axis_name="s")

    @functools.partial(
        pl.kernel,
        out_type=jax.ShapeDtypeStruct((2 * np_nodes, channels), jnp.float32),
        mesh=mesh,
        scratch_types=[
            pltpu.VMEM((nct, _K), jnp.int32),
            pltpu.VMEM((nct, _K), jnp.int32),
            pltpu.VMEM((_K, channels), jnp.float32),
            pltpu.VMEM_SHARED((np_nodes, channels), jnp.float32),
            pltpu.SemaphoreType.DMA,
        ],
    )
    def spmm(h_hbm, src_hbm, dst_hbm, zeros_hbm, out_hbm,
             src_v, dst_v, rows_v, acc, sem):
        cid = lax.axis_index("c")
        sid = lax.axis_index("s")
        tid = cid * _NS + sid
        pltpu.sync_copy(src_hbm.at[pl.ds(tid * nct, nct)], src_v)
        pltpu.sync_copy(dst_hbm.at[pl.ds(tid * nct, nct)], dst_v)
        pltpu.sync_copy(zeros_hbm.at[pl.ds(sid * rpt, rpt)],
                        acc.at[pl.ds(sid * rpt, rpt)])
        plsc.subcore_barrier()

        def body(j, carry):
            pltpu.async_copy(h_hbm.at[src_v.at[j]], rows_v, sem).wait()
            pltpu.sync_copy(rows_v, acc.at[dst_v.at[j]], add=True)
            return carry

        lax.fori_loop(0, nct, body, 0)
        plsc.subcore_barrier()
        pltpu.sync_copy(acc.at[pl.ds(sid * rpt, rpt)],
                        out_hbm.at[pl.ds(cid * np_nodes + sid * rpt, rpt)])

    return spmm(h, src2d, dst2d, zeros)


def _mm2_body(p0, p1, w0, w1, o):
    s = p0[...] + p1[...]
    hid = jnp.maximum(jnp.dot(s, w0[...], preferred_element_type=jnp.float32), 0.0)
    o[...] = jnp.dot(hid, w1[...], preferred_element_type=jnp.float32)


def _relu_body(q0, q1, o):
    o[...] = jnp.maximum(q0[...] + q1[...], 0.0)


def _mm_softmax_body(r0, r1, w2, o):
    s = jnp.dot(r0[...] + r1[...], w2[...], preferred_element_type=jnp.float32)
    m = jnp.max(s, axis=-1, keepdims=True)
    e = jnp.exp(s - m)
    o[...] = e / jnp.sum(e, axis=-1, keepdims=True)


def kernel(x, edge_index, W0, W1, W2):
    n, d_feat = x.shape
    e = edge_index.shape[1]
    c0 = W0.shape[1]
    c1 = W1.shape[1]
    ncls = W2.shape[1]

    npad = ((n + 127) // 128) * 128
    src2d = edge_index[0].reshape(e // _K, _K)
    dst2d = edge_index[1].reshape(e // _K, _K)
    zeros_f = jnp.zeros((npad, d_feat), jnp.float32)

    blk = 1000
    grid = (n // blk,)

    agg0 = _spmm_partials(x, src2d, dst2d, zeros_f,
                          n_nodes=n, channels=d_feat)

    t1 = pl.pallas_call(
        _mm2_body,
        grid=grid,
        in_specs=[
            pl.BlockSpec((blk, d_feat), lambda i: (i, 0)),
            pl.BlockSpec((blk, d_feat), lambda i: (i, 0)),
            pl.BlockSpec((d_feat, c0), lambda i: (0, 0)),
            pl.BlockSpec((c0, c1), lambda i: (0, 0)),
        ],
        out_specs=pl.BlockSpec((blk, c1), lambda i: (i, 0)),
        out_shape=jax.ShapeDtypeStruct((n, c1), jnp.float32),
    )(agg0[:n], agg0[npad:npad + n], W0, W1)

    agg1 = _spmm_partials(t1, src2d, dst2d, zeros_f,
                          n_nodes=n, channels=c1)

    h1 = pl.pallas_call(
        _relu_body,
        grid=grid,
        in_specs=[
            pl.BlockSpec((blk, c1), lambda i: (i, 0)),
            pl.BlockSpec((blk, c1), lambda i: (i, 0)),
        ],
        out_specs=pl.BlockSpec((blk, c1), lambda i: (i, 0)),
        out_shape=jax.ShapeDtypeStruct((n, c1), jnp.float32),
    )(agg1[:n], agg1[npad:npad + n])

    agg2 = _spmm_partials(h1, src2d, dst2d, zeros_f,
                          n_nodes=n, channels=c1)

    out = pl.pallas_call(
        _mm_softmax_body,
        grid=grid,
        in_specs=[
            pl.BlockSpec((blk, c1), lambda i: (i, 0)),
            pl.BlockSpec((blk, c1), lambda i: (i, 0)),
            pl.BlockSpec((c1, ncls), lambda i: (0, 0)),
        ],
        out_specs=pl.BlockSpec((blk, ncls), lambda i: (i, 0)),
        out_shape=jax.ShapeDtypeStruct((n, ncls), jnp.float32),
    )(agg2[:n], agg2[npad:npad + n], W2)

    return out
